# baseline (speedup 1.0000x reference)
import functools

import jax
import jax.numpy as jnp
from jax import lax
from jax.experimental import pallas as pl
from jax.experimental.pallas import tpu as pltpu
from jax.experimental.pallas import tpu_sc as plsc

NU, NI, NT = 5000, 4000, 1000
NN = NU + NI + NT
D = 128
E = 320000

NTILES = 32
CPT = D // NTILES
TBLW = NN * CPT
ACCW = NU * CPT
CHUNK = 2560
NB = CHUNK // 16
NCH = E // CHUNK


def _tc_pre_body(c_ref, o_ref, w1t_ref, b1_ref, w2t_ref, b2_ref,
                 eh_ref, p_ref, oo_ref):
    c = c_ref[...]
    h = jnp.dot(c, w1t_ref[...], preferred_element_type=jnp.float32)
    h = jnp.maximum(h + b1_ref[...], 0.0)
    h = jnp.dot(h, w2t_ref[...], preferred_element_type=jnp.float32)
    h = h + b2_ref[...]
    md = jnp.max(h, axis=0, keepdims=True)
    eh = jnp.exp(h - md)
    eh_ref[...] = eh
    p_ref[...] = eh * c
    oo_ref[...] = jnp.maximum(o_ref[...], 0.0)


_tc_pre = pl.pallas_call(
    _tc_pre_body,
    out_shape=[
        jax.ShapeDtypeStruct((NN, D), jnp.float32),
        jax.ShapeDtypeStruct((NN, D), jnp.float32),
        jax.ShapeDtypeStruct((NN, D), jnp.float32),
    ],
)


def _tc_post_body(num_ref, den_ref, offm_ref, emb_ref, off_ref):
    num = num_ref[...]
    den = den_ref[...]
    emb = num / (den + 1e-16)
    n2 = jnp.sum(emb * emb, axis=1, keepdims=True)
    emb_ref[...] = emb / jnp.maximum(jnp.sqrt(n2), 1e-12)
    off_ref[...] = jnp.maximum(offm_ref[...], 0.0)


_tc_post = pl.pallas_call(
    _tc_post_body,
    out_shape=[
        jax.ShapeDtypeStruct((NU, D), jnp.float32),
        jax.ShapeDtypeStruct((NU, D), jnp.float32),
    ],
)


def _sc_body(eh_hbm, p_hbm, o_hbm, h1_hbm, t1_hbm, h2_hbm, t2_hbm,
             den_hbm, num_hbm, off_hbm,
             tbl_a, tbl_b, acc_a, acc_b, hb0, tb0, hb1, tb1, sem0, sem1):
    wid = lax.axis_index("s") * 2 + lax.axis_index("c")

    def zero(acc, words):
        zv = jnp.zeros((16,), jnp.float32)

        @plsc.parallel_loop(0, words // 16, unroll=2)
        def zb(i):
            acc[pl.ds(i * 16, 16)] = zv

    def _start(hsrc, tsrc, hdst, tdst, sem, ch):
        off = pl.multiple_of(ch * CHUNK, CHUNK)
        pltpu.make_async_copy(hsrc.at[pl.ds(off, CHUNK)], hdst, sem).start()
        pltpu.make_async_copy(tsrc.at[pl.ds(off, CHUNK)], tdst, sem).start()

    def _wait(hsrc, tsrc, hdst, tdst, sem, ch):
        off = pl.multiple_of(ch * CHUNK, CHUNK)
        pltpu.make_async_copy(hsrc.at[pl.ds(off, CHUNK)], hdst, sem).wait()
        pltpu.make_async_copy(tsrc.at[pl.ds(off, CHUNK)], tdst, sem).wait()

    def stream_edges(hsrc, tsrc, process):
        _start(hsrc, tsrc, hb0, tb0, sem0, 0)

        def body(g, carry):
            ch = g * 2

            @pl.when(ch + 1 < NCH)
            def _():
                _start(hsrc, tsrc, hb1, tb1, sem1, ch + 1)

            _wait(hsrc, tsrc, hb0, tb0, sem0, ch)
            process(hb0, tb0)

            @pl.when(ch + 2 < NCH)
            def _():
                _start(hsrc, tsrc, hb0, tb0, sem0, ch + 2)

            @pl.when(ch + 1 < NCH)
            def _():
                _wait(hsrc, tsrc, hb1, tb1, sem1, ch + 1)
                process(hb1, tb1)

            return carry

        lax.fori_loop(0, (NCH + 1) // 2, body, 0)

    def sum_process(hbuf, tbuf):
        @plsc.parallel_loop(0, NB, unroll=4)
        def batch(i):
            heads = hbuf[pl.ds(i * 16, 16)]
            tails = tbuf[pl.ds(i * 16, 16)]
            msk = heads < NU
            hb = jnp.where(msk, heads, 0) * CPT
            tb = tails * CPT
            for c in range(CPT):
                v = plsc.load_gather(tbl_a, [tb + c])
                plsc.addupdate_scatter(acc_a, [hb + c], v, mask=msk)
                w = plsc.load_gather(tbl_b, [tb + c])
                plsc.addupdate_scatter(acc_b, [hb + c], w, mask=msk)

    def make_max_process(tail_lo):
        def max_process(hbuf, tbuf):
            def batch(i, c2):
                heads = hbuf[pl.ds(i * 16, 16)]
                tails = tbuf[pl.ds(i * 16, 16)]
                msk = heads < NU
                if tail_lo:
                    msk = msk & (tails >= tail_lo)
                hb = jnp.where(msk, heads, 0) * CPT
                tb = tails * CPT
                idxs = [hb + c for c in range(CPT)]
                vals = [plsc.load_gather(tbl_a, [tb + c])
                        for c in range(CPT)]
                pend = []
                for c in range(CPT):
                    cur = plsc.load_gather(acc_a, [idxs[c]])
                    need = msk & (vals[c] > cur)
                    plsc.store_scatter(acc_a, [idxs[c]], vals[c], mask=need)
                    pend.append(need)
                lost = []
                for c in range(CPT):
                    cur2 = plsc.load_gather(acc_a, [idxs[c]])
                    lost.append(pend[c] & (cur2 < vals[c]))

                def wcond(st):
                    return jnp.any((st[0] | st[1]) | (st[2] | st[3]))

                def wbody(st):
                    out = []
                    for c in range(CPT):
                        cur = plsc.load_gather(acc_a, [idxs[c]])
                        need = st[c] & (vals[c] > cur)
                        plsc.store_scatter(acc_a, [idxs[c]], vals[c],
                                           mask=need)
                        cur2 = plsc.load_gather(acc_a, [idxs[c]])
                        out.append(need & (cur2 < vals[c]))
                    return tuple(out)

                lax.while_loop(wcond, wbody, tuple(lost))
                return c2

            lax.fori_loop(0, NB, batch, 0)

        return max_process

    with jax.named_scope("sc_sums"):
        pltpu.sync_copy(eh_hbm.at[wid], tbl_a)
        pltpu.sync_copy(p_hbm.at[wid], tbl_b)
        zero(acc_a, ACCW)
        zero(acc_b, ACCW)
        stream_edges(h1_hbm, t1_hbm, sum_process)
        pltpu.sync_copy(acc_a, den_hbm.at[wid])
        pltpu.sync_copy(acc_b, num_hbm.at[wid])

    with jax.named_scope("sc_max"):
        pltpu.sync_copy(o_hbm.at[wid], tbl_a)
        zero(acc_a, ACCW)
        stream_edges(h1_hbm, t1_hbm, make_max_process(NU))
        stream_edges(h2_hbm, t2_hbm, make_max_process(0))
        pltpu.sync_copy(acc_a, off_hbm.at[wid])


_sc_call = pl.kernel(
    _sc_body,
    out_type=(
        jax.ShapeDtypeStruct((NTILES, ACCW), jnp.float32),
        jax.ShapeDtypeStruct((NTILES, ACCW), jnp.float32),
        jax.ShapeDtypeStruct((NTILES, ACCW), jnp.float32),
    ),
    mesh=plsc.VectorSubcoreMesh(core_axis_name="c", subcore_axis_name="s"),
    compiler_params=pltpu.CompilerParams(needs_layout_passes=False),
    scratch_types=[
        pltpu.VMEM((TBLW,), jnp.float32),
        pltpu.VMEM((TBLW,), jnp.float32),
        pltpu.VMEM((ACCW,), jnp.float32),
        pltpu.VMEM((ACCW,), jnp.float32),
        pltpu.VMEM((CHUNK,), jnp.int32),
        pltpu.VMEM((CHUNK,), jnp.int32),
        pltpu.VMEM((CHUNK,), jnp.int32),
        pltpu.VMEM((CHUNK,), jnp.int32),
        pltpu.SemaphoreType.DMA,
        pltpu.SemaphoreType.DMA,
    ],
)


def _slab(x):
    return x.reshape(NN, NTILES, CPT).transpose(1, 0, 2).reshape(NTILES, TBLW)


def _unslab(x):
    return x.reshape(NTILES, NU, CPT).transpose(1, 0, 2).reshape(NU, D)


def kernel(user_center, user_offset, item_center, item_offset, tag_center,
           tag_offset, graph1, graph2, visit_time, Wc1, bc1, Wc2, bc2,
           Wt1, bt1, Wt2, bt2):
    all_center = jnp.concatenate([user_center, item_center, tag_center], axis=0)
    all_offset = jnp.concatenate([user_offset, item_offset, tag_offset], axis=0)

    eh, p, oo = _tc_pre(all_center, all_offset,
                        Wc1.T, bc1.reshape(1, D),
                        Wc2.T, bc2.reshape(1, D))

    den_s, num_s, off_s = _sc_call(
        _slab(eh), _slab(p), _slab(oo),
        graph1[0], graph1[1], graph2[0], graph2[1])

    emb, off = _tc_post(_unslab(num_s), _unslab(den_s), _unslab(off_s))
    return emb, off


# speedup vs baseline: 1.2022x; 1.2022x over previous
import functools

import jax
import jax.numpy as jnp
from jax import lax
from jax.experimental import pallas as pl
from jax.experimental.pallas import tpu as pltpu
from jax.experimental.pallas import tpu_sc as plsc

NU, NI, NT = 5000, 4000, 1000
NN = NU + NI + NT
D = 128
E = 320000

NTILES = 32
CPT = D // NTILES
TBLW = NN * CPT
ACCW = NU * CPT
CHUNK = 2560
NB = CHUNK // 16
NCH = E // CHUNK


def _tc_pre_body(c_ref, o_ref, w1t_ref, b1_ref, w2t_ref, b2_ref,
                 eh_ref, p_ref, oo_ref):
    c = c_ref[...]
    h = jnp.dot(c, w1t_ref[...], preferred_element_type=jnp.float32)
    h = jnp.maximum(h + b1_ref[...], 0.0)
    h = jnp.dot(h, w2t_ref[...], preferred_element_type=jnp.float32)
    h = h + b2_ref[...]
    md = jnp.max(h, axis=0, keepdims=True)
    eh = jnp.exp(h - md)
    eh_ref[...] = eh
    p_ref[...] = eh * c
    oo_ref[...] = jnp.maximum(o_ref[...], 0.0)


_tc_pre = pl.pallas_call(
    _tc_pre_body,
    out_shape=[
        jax.ShapeDtypeStruct((NN, D), jnp.float32),
        jax.ShapeDtypeStruct((NN, D), jnp.float32),
        jax.ShapeDtypeStruct((NN, D), jnp.float32),
    ],
)


def _tc_post_body(num_ref, den_ref, offm_ref, emb_ref, off_ref):
    num = num_ref[...]
    den = den_ref[...]
    emb = num / (den + 1e-16)
    n2 = jnp.sum(emb * emb, axis=1, keepdims=True)
    emb_ref[...] = emb / jnp.maximum(jnp.sqrt(n2), 1e-12)
    off_ref[...] = jnp.maximum(offm_ref[...], 0.0)


_tc_post = pl.pallas_call(
    _tc_post_body,
    out_shape=[
        jax.ShapeDtypeStruct((NU, D), jnp.float32),
        jax.ShapeDtypeStruct((NU, D), jnp.float32),
    ],
)


def _sc_body(eh_hbm, p_hbm, o_hbm, h1_hbm, t1_hbm, h2_hbm, t2_hbm,
             den_hbm, num_hbm, off_hbm,
             tbl_a, tbl_b, acc_a, acc_b, hb0, tb0, hb1, tb1, sem0, sem1):
    wid = lax.axis_index("s") * 2 + lax.axis_index("c")

    def zero(acc, words):
        zv = jnp.zeros((16,), jnp.float32)

        @plsc.parallel_loop(0, words // 16, unroll=2)
        def zb(i):
            acc[pl.ds(i * 16, 16)] = zv

    def _start(hsrc, tsrc, hdst, tdst, sem, ch):
        off = pl.multiple_of(ch * CHUNK, CHUNK)
        pltpu.make_async_copy(hsrc.at[pl.ds(off, CHUNK)], hdst, sem).start()
        pltpu.make_async_copy(tsrc.at[pl.ds(off, CHUNK)], tdst, sem).start()

    def _wait(hsrc, tsrc, hdst, tdst, sem, ch):
        off = pl.multiple_of(ch * CHUNK, CHUNK)
        pltpu.make_async_copy(hsrc.at[pl.ds(off, CHUNK)], hdst, sem).wait()
        pltpu.make_async_copy(tsrc.at[pl.ds(off, CHUNK)], tdst, sem).wait()

    def stream_edges(hsrc, tsrc, process):
        _start(hsrc, tsrc, hb0, tb0, sem0, 0)

        def body(g, carry):
            ch = g * 2

            @pl.when(ch + 1 < NCH)
            def _():
                _start(hsrc, tsrc, hb1, tb1, sem1, ch + 1)

            _wait(hsrc, tsrc, hb0, tb0, sem0, ch)
            process(hb0, tb0)

            @pl.when(ch + 2 < NCH)
            def _():
                _start(hsrc, tsrc, hb0, tb0, sem0, ch + 2)

            @pl.when(ch + 1 < NCH)
            def _():
                _wait(hsrc, tsrc, hb1, tb1, sem1, ch + 1)
                process(hb1, tb1)

            return carry

        lax.fori_loop(0, (NCH + 1) // 2, body, 0)

    def sum_process(hbuf, tbuf):
        @plsc.parallel_loop(0, NB, unroll=4)
        def batch(i):
            heads = hbuf[pl.ds(i * 16, 16)]
            tails = tbuf[pl.ds(i * 16, 16)]
            msk = heads < NU
            hb = jnp.where(msk, heads, 0) * CPT
            tb = tails * CPT
            for c in range(CPT):
                v = plsc.load_gather(tbl_a, [tb + c])
                plsc.addupdate_scatter(acc_a, [hb + c], v, mask=msk)
                w = plsc.load_gather(tbl_b, [tb + c])
                plsc.addupdate_scatter(acc_b, [hb + c], w, mask=msk)

    def make_max_process(tail_lo):
        ILV = 2

        def max_process(hbuf, tbuf):
            def pair(i, c2):
                datas = []
                for k in range(ILV):
                    j = i * ILV + k
                    heads = hbuf[pl.ds(j * 16, 16)]
                    tails = tbuf[pl.ds(j * 16, 16)]
                    msk = heads < NU
                    if tail_lo:
                        msk = msk & (tails >= tail_lo)
                    hb = jnp.where(msk, heads, 0) * CPT
                    tb = tails * CPT
                    idxs = [hb + c for c in range(CPT)]
                    vals = [plsc.load_gather(tbl_a, [tb + c])
                            for c in range(CPT)]
                    datas.append((msk, idxs, vals))

                def step(st):
                    needs = []
                    n = 0
                    for (msk, idxs, vals) in datas:
                        for c in range(CPT):
                            cur = plsc.load_gather(acc_a, [idxs[c]])
                            gate = msk if st is None else st[n]
                            need = gate & (vals[c] > cur)
                            plsc.store_scatter(acc_a, [idxs[c]], vals[c],
                                               mask=need)
                            needs.append(need)
                            n += 1
                    out = []
                    n = 0
                    for (msk, idxs, vals) in datas:
                        for c in range(CPT):
                            cur2 = plsc.load_gather(acc_a, [idxs[c]])
                            out.append(needs[n] & (cur2 < vals[c]))
                            n += 1
                    return tuple(out)

                lost = step(None)

                def wcond(st):
                    m = st[0]
                    for s in st[1:]:
                        m = m | s
                    return jnp.any(m)

                lax.while_loop(wcond, step, lost)
                return c2

            lax.fori_loop(0, NB // ILV, pair, 0)

        return max_process

    with jax.named_scope("sc_sums"):
        pltpu.sync_copy(eh_hbm.at[wid], tbl_a)
        pltpu.sync_copy(p_hbm.at[wid], tbl_b)
        zero(acc_a, ACCW)
        zero(acc_b, ACCW)
        stream_edges(h1_hbm, t1_hbm, sum_process)
        pltpu.sync_copy(acc_a, den_hbm.at[wid])
        pltpu.sync_copy(acc_b, num_hbm.at[wid])

    with jax.named_scope("sc_max"):
        pltpu.sync_copy(o_hbm.at[wid], tbl_a)
        zero(acc_a, ACCW)
        stream_edges(h1_hbm, t1_hbm, make_max_process(NU))
        stream_edges(h2_hbm, t2_hbm, make_max_process(0))
        pltpu.sync_copy(acc_a, off_hbm.at[wid])


_sc_call = pl.kernel(
    _sc_body,
    out_type=(
        jax.ShapeDtypeStruct((NTILES, ACCW), jnp.float32),
        jax.ShapeDtypeStruct((NTILES, ACCW), jnp.float32),
        jax.ShapeDtypeStruct((NTILES, ACCW), jnp.float32),
    ),
    mesh=plsc.VectorSubcoreMesh(core_axis_name="c", subcore_axis_name="s"),
    compiler_params=pltpu.CompilerParams(needs_layout_passes=False),
    scratch_types=[
        pltpu.VMEM((TBLW,), jnp.float32),
        pltpu.VMEM((TBLW,), jnp.float32),
        pltpu.VMEM((ACCW,), jnp.float32),
        pltpu.VMEM((ACCW,), jnp.float32),
        pltpu.VMEM((CHUNK,), jnp.int32),
        pltpu.VMEM((CHUNK,), jnp.int32),
        pltpu.VMEM((CHUNK,), jnp.int32),
        pltpu.VMEM((CHUNK,), jnp.int32),
        pltpu.SemaphoreType.DMA,
        pltpu.SemaphoreType.DMA,
    ],
)


def _slab(x):
    return x.reshape(NN, NTILES, CPT).transpose(1, 0, 2).reshape(NTILES, TBLW)


def _unslab(x):
    return x.reshape(NTILES, NU, CPT).transpose(1, 0, 2).reshape(NU, D)


def kernel(user_center, user_offset, item_center, item_offset, tag_center,
           tag_offset, graph1, graph2, visit_time, Wc1, bc1, Wc2, bc2,
           Wt1, bt1, Wt2, bt2):
    all_center = jnp.concatenate([user_center, item_center, tag_center], axis=0)
    all_offset = jnp.concatenate([user_offset, item_offset, tag_offset], axis=0)

    eh, p, oo = _tc_pre(all_center, all_offset,
                        Wc1.T, bc1.reshape(1, D),
                        Wc2.T, bc2.reshape(1, D))

    den_s, num_s, off_s = _sc_call(
        _slab(eh), _slab(p), _slab(oo),
        graph1[0], graph1[1], graph2[0], graph2[1])

    emb, off = _tc_post(_unslab(num_s), _unslab(den_s), _unslab(off_s))
    return emb, off


# speedup vs baseline: 1.2024x; 1.0002x over previous
import functools

import jax
import jax.numpy as jnp
from jax import lax
from jax.experimental import pallas as pl
from jax.experimental.pallas import tpu as pltpu
from jax.experimental.pallas import tpu_sc as plsc

NU, NI, NT = 5000, 4000, 1000
NN = NU + NI + NT
D = 128
E = 320000

NTILES = 32
CPT = D // NTILES
TBLW = NN * CPT
ACCW = NU * CPT
CHUNK = 2560
NB = CHUNK // 16
NCH = E // CHUNK


def _tc_pre_body(c_ref, o_ref, w1t_ref, b1_ref, w2t_ref, b2_ref,
                 eh_ref, p_ref, oo_ref):
    c = c_ref[...]
    h = jnp.dot(c, w1t_ref[...], preferred_element_type=jnp.float32)
    h = jnp.maximum(h + b1_ref[...], 0.0)
    h = jnp.dot(h, w2t_ref[...], preferred_element_type=jnp.float32)
    h = h + b2_ref[...]
    md = jnp.max(h, axis=0, keepdims=True)
    eh = jnp.exp(h - md)
    eh_ref[...] = eh
    p_ref[...] = eh * c
    oo_ref[...] = jnp.maximum(o_ref[...], 0.0)


_tc_pre = pl.pallas_call(
    _tc_pre_body,
    out_shape=[
        jax.ShapeDtypeStruct((NN, D), jnp.float32),
        jax.ShapeDtypeStruct((NN, D), jnp.float32),
        jax.ShapeDtypeStruct((NN, D), jnp.float32),
    ],
)


def _tc_post_body(num_ref, den_ref, offm_ref, emb_ref, off_ref):
    num = num_ref[...]
    den = den_ref[...]
    emb = num / (den + 1e-16)
    n2 = jnp.sum(emb * emb, axis=1, keepdims=True)
    emb_ref[...] = emb / jnp.maximum(jnp.sqrt(n2), 1e-12)
    off_ref[...] = jnp.maximum(offm_ref[...], 0.0)


_tc_post = pl.pallas_call(
    _tc_post_body,
    out_shape=[
        jax.ShapeDtypeStruct((NU, D), jnp.float32),
        jax.ShapeDtypeStruct((NU, D), jnp.float32),
    ],
)


def _sc_body(eh_hbm, p_hbm, o_hbm, g1_hbm, g2_hbm,
             den_hbm, num_hbm, off_hbm,
             tbl_a, tbl_b, acc_a, acc_b, eb0, eb1, sem0, sem1):
    wid = lax.axis_index("s") * 2 + lax.axis_index("c")

    def zero(acc, words):
        zv = jnp.zeros((16,), jnp.float32)

        @plsc.parallel_loop(0, words // 16, unroll=2)
        def zb(i):
            acc[pl.ds(i * 16, 16)] = zv

    def _start(src, dst, sem, ch):
        pltpu.make_async_copy(src.at[ch], dst, sem).start()

    def _wait(src, dst, sem, ch):
        pltpu.make_async_copy(src.at[ch], dst, sem).wait()

    def stream_edges(src, process):
        _start(src, eb0, sem0, 0)

        def body(g, carry):
            ch = g * 2

            @pl.when(ch + 1 < NCH)
            def _():
                _start(src, eb1, sem1, ch + 1)

            _wait(src, eb0, sem0, ch)
            process(eb0)

            @pl.when(ch + 2 < NCH)
            def _():
                _start(src, eb0, sem0, ch + 2)

            @pl.when(ch + 1 < NCH)
            def _():
                _wait(src, eb1, sem1, ch + 1)
                process(eb1)

            return carry

        lax.fori_loop(0, (NCH + 1) // 2, body, 0)

    def sum_process(ebuf):
        @plsc.parallel_loop(0, NB, unroll=4)
        def batch(i):
            heads = ebuf[pl.ds(i * 16, 16)]
            tails = ebuf[pl.ds(CHUNK + i * 16, 16)]
            msk = heads < NU
            hb = jnp.where(msk, heads, 0) * CPT
            tb = tails * CPT
            for c in range(CPT):
                v = plsc.load_gather(tbl_a, [tb + c])
                plsc.addupdate_scatter(acc_a, [hb + c], v, mask=msk)
                w = plsc.load_gather(tbl_b, [tb + c])
                plsc.addupdate_scatter(acc_b, [hb + c], w, mask=msk)

    def make_max_process(tail_lo):
        ILV = 2

        def max_process(ebuf):
            def pair(i, c2):
                datas = []
                for k in range(ILV):
                    j = i * ILV + k
                    heads = ebuf[pl.ds(j * 16, 16)]
                    tails = ebuf[pl.ds(CHUNK + j * 16, 16)]
                    msk = heads < NU
                    if tail_lo:
                        msk = msk & (tails >= tail_lo)
                    hb = jnp.where(msk, heads, 0) * CPT
                    tb = tails * CPT
                    idxs = [hb + c for c in range(CPT)]
                    vals = [plsc.load_gather(tbl_a, [tb + c])
                            for c in range(CPT)]
                    datas.append((msk, idxs, vals))

                def step(st):
                    needs = []
                    n = 0
                    for (msk, idxs, vals) in datas:
                        for c in range(CPT):
                            cur = plsc.load_gather(acc_a, [idxs[c]])
                            gate = msk if st is None else st[n]
                            need = gate & (vals[c] > cur)
                            plsc.store_scatter(acc_a, [idxs[c]], vals[c],
                                               mask=need)
                            needs.append(need)
                            n += 1
                    out = []
                    n = 0
                    for (msk, idxs, vals) in datas:
                        for c in range(CPT):
                            cur2 = plsc.load_gather(acc_a, [idxs[c]])
                            out.append(needs[n] & (cur2 < vals[c]))
                            n += 1
                    return tuple(out)

                lost = step(None)

                def wcond(st):
                    m = st[0]
                    for s in st[1:]:
                        m = m | s
                    return jnp.any(m)

                lax.while_loop(wcond, step, lost)
                return c2

            lax.fori_loop(0, NB // ILV, pair, 0)

        return max_process

    with jax.named_scope("sc_sums"):
        pltpu.sync_copy(eh_hbm.at[wid], tbl_a)
        pltpu.sync_copy(p_hbm.at[wid], tbl_b)
        zero(acc_a, ACCW)
        zero(acc_b, ACCW)
        stream_edges(g1_hbm, sum_process)
        pltpu.sync_copy(acc_a, den_hbm.at[wid])
        pltpu.sync_copy(acc_b, num_hbm.at[wid])

    with jax.named_scope("sc_max"):
        pltpu.sync_copy(o_hbm.at[wid], tbl_a)
        zero(acc_a, ACCW)
        stream_edges(g1_hbm, make_max_process(NU))
        stream_edges(g2_hbm, make_max_process(0))
        pltpu.sync_copy(acc_a, off_hbm.at[wid])


_sc_call = pl.kernel(
    _sc_body,
    out_type=(
        jax.ShapeDtypeStruct((NTILES, ACCW), jnp.float32),
        jax.ShapeDtypeStruct((NTILES, ACCW), jnp.float32),
        jax.ShapeDtypeStruct((NTILES, ACCW), jnp.float32),
    ),
    mesh=plsc.VectorSubcoreMesh(core_axis_name="c", subcore_axis_name="s"),
    compiler_params=pltpu.CompilerParams(needs_layout_passes=False),
    scratch_types=[
        pltpu.VMEM((TBLW,), jnp.float32),
        pltpu.VMEM((TBLW,), jnp.float32),
        pltpu.VMEM((ACCW,), jnp.float32),
        pltpu.VMEM((ACCW,), jnp.float32),
        pltpu.VMEM((2 * CHUNK,), jnp.int32),
        pltpu.VMEM((2 * CHUNK,), jnp.int32),
        pltpu.SemaphoreType.DMA,
        pltpu.SemaphoreType.DMA,
    ],
)


def _slab(x):
    return x.reshape(NN, NTILES, CPT).transpose(1, 0, 2).reshape(NTILES, TBLW)


def _unslab(x):
    return x.reshape(NTILES, NU, CPT).transpose(1, 0, 2).reshape(NU, D)


def _pack_edges(g):
    return (g.reshape(2, NCH, CHUNK).transpose(1, 0, 2)
             .reshape(NCH, 2 * CHUNK))


def kernel(user_center, user_offset, item_center, item_offset, tag_center,
           tag_offset, graph1, graph2, visit_time, Wc1, bc1, Wc2, bc2,
           Wt1, bt1, Wt2, bt2):
    all_center = jnp.concatenate([user_center, item_center, tag_center], axis=0)
    all_offset = jnp.concatenate([user_offset, item_offset, tag_offset], axis=0)

    eh, p, oo = _tc_pre(all_center, all_offset,
                        Wc1.T, bc1.reshape(1, D),
                        Wc2.T, bc2.reshape(1, D))

    den_s, num_s, off_s = _sc_call(
        _slab(eh), _slab(p), _slab(oo),
        _pack_edges(graph1), _pack_edges(graph2))

    emb, off = _tc_post(_unslab(num_s), _unslab(den_s), _unslab(off_s))
    return emb, off


# speedup vs baseline: 1.8134x; 1.5081x over previous
import functools

import jax
import jax.numpy as jnp
from jax import lax
from jax.experimental import pallas as pl
from jax.experimental.pallas import tpu as pltpu
from jax.experimental.pallas import tpu_sc as plsc

NU, NI, NT = 5000, 4000, 1000
NN = NU + NI + NT
D = 128
E = 320000

NTILES = 32
CPT = D // NTILES
TBLW = NN * CPT
ACCW = NU * CPT
CHUNK = 2560
NB = CHUNK // 16
NCH = E // CHUNK


def _tc_pre_body(c_ref, o_ref, w1t_ref, b1_ref, w2t_ref, b2_ref,
                 eh_ref, p_ref, oo_ref):
    c = c_ref[...]
    h = jnp.dot(c, w1t_ref[...], preferred_element_type=jnp.float32)
    h = jnp.maximum(h + b1_ref[...], 0.0)
    h = jnp.dot(h, w2t_ref[...], preferred_element_type=jnp.float32)
    h = h + b2_ref[...]
    md = jnp.max(h, axis=0, keepdims=True)
    eh = jnp.exp(h - md)
    eh_ref[...] = eh
    p_ref[...] = eh * c
    oo_ref[...] = jnp.maximum(o_ref[...], 0.0)


_tc_pre = pl.pallas_call(
    _tc_pre_body,
    out_shape=[
        jax.ShapeDtypeStruct((NN, D), jnp.float32),
        jax.ShapeDtypeStruct((NN, D), jnp.float32),
        jax.ShapeDtypeStruct((NN, D), jnp.float32),
    ],
)


def _tc_post_body(num_ref, den_ref, offm_ref, emb_ref, off_ref):
    num = num_ref[...]
    den = den_ref[...]
    emb = num / (den + 1e-16)
    n2 = jnp.sum(emb * emb, axis=1, keepdims=True)
    emb_ref[...] = emb / jnp.maximum(jnp.sqrt(n2), 1e-12)
    off_ref[...] = jnp.maximum(offm_ref[...], 0.0)


_tc_post = pl.pallas_call(
    _tc_post_body,
    out_shape=[
        jax.ShapeDtypeStruct((NU, D), jnp.float32),
        jax.ShapeDtypeStruct((NU, D), jnp.float32),
    ],
)


def _sc_body(eh_hbm, p_hbm, o_hbm, g1_hbm, g2_hbm,
             den_hbm, num_hbm, off_hbm,
             tbl_a, tbl_b, acc_a, acc_b, eb0, eb1, sem0, sem1):
    wid = lax.axis_index("s") * 2 + lax.axis_index("c")

    def zero(acc, words):
        zv = jnp.zeros((16,), jnp.float32)

        @plsc.parallel_loop(0, words // 16, unroll=2)
        def zb(i):
            acc[pl.ds(i * 16, 16)] = zv

    def _start(src, dst, sem, ch):
        pltpu.make_async_copy(src.at[ch], dst, sem).start()

    def _wait(src, dst, sem, ch):
        pltpu.make_async_copy(src.at[ch], dst, sem).wait()

    def stream_edges(src, process):
        _start(src, eb0, sem0, 0)

        def body(g, carry):
            ch = g * 2

            @pl.when(ch + 1 < NCH)
            def _():
                _start(src, eb1, sem1, ch + 1)

            _wait(src, eb0, sem0, ch)
            process(eb0)

            @pl.when(ch + 2 < NCH)
            def _():
                _start(src, eb0, sem0, ch + 2)

            @pl.when(ch + 1 < NCH)
            def _():
                _wait(src, eb1, sem1, ch + 1)
                process(eb1)

            return carry

        lax.fori_loop(0, (NCH + 1) // 2, body, 0)

    def sum_process(ebuf):
        @plsc.parallel_loop(0, NB, unroll=4)
        def batch(i):
            heads = ebuf[pl.ds(i * 16, 16)]
            tails = ebuf[pl.ds(CHUNK + i * 16, 16)]
            msk = heads < NU
            hb = jnp.where(msk, heads, 0) * CPT
            tb = tails * CPT
            for c in range(CPT):
                v = plsc.load_gather(tbl_a, [tb + c])
                plsc.addupdate_scatter(acc_a, [hb + c], v, mask=msk)
                w = plsc.load_gather(tbl_b, [tb + c])
                plsc.addupdate_scatter(acc_b, [hb + c], w, mask=msk)

    def make_max_process(tail_lo):
        ILV = 2

        def max_process(ebuf):
            iota = lax.broadcasted_iota(jnp.int32, (16,), 0)

            def compact(i, cnt):
                heads = ebuf[pl.ds(i * 16, 16)]
                tails = ebuf[pl.ds(CHUNK + i * 16, 16)]
                msk = heads < NU
                if tail_lo:
                    msk = msk & (tails >= tail_lo)
                pos = cnt + plsc.cumsum(jnp.where(msk, 1, 0)) - 1
                pos = jnp.where(msk, pos, 0)
                plsc.store_scatter(ebuf, [pos], heads, mask=msk)
                plsc.store_scatter(ebuf, [pos + CHUNK], tails, mask=msk)
                return cnt + plsc.all_reduce_population_count(msk)

            cnt = lax.fori_loop(0, NB, compact, jnp.zeros((16,), jnp.int32))

            def pair(i):
                datas = []
                for k in range(ILV):
                    j = i * ILV + k
                    heads = ebuf[pl.ds(j * 16, 16)]
                    tails = ebuf[pl.ds(CHUNK + j * 16, 16)]
                    msk = (j * 16 + iota) < cnt
                    hb = jnp.where(msk, heads, 0) * CPT
                    tb = tails * CPT
                    idxs = [hb + c for c in range(CPT)]
                    vals = [plsc.load_gather(tbl_a, [tb + c])
                            for c in range(CPT)]
                    datas.append((msk, idxs, vals))

                def step(st):
                    needs = []
                    n = 0
                    for (msk, idxs, vals) in datas:
                        for c in range(CPT):
                            cur = plsc.load_gather(acc_a, [idxs[c]])
                            gate = msk if st is None else st[n]
                            need = gate & (vals[c] > cur)
                            plsc.store_scatter(acc_a, [idxs[c]], vals[c],
                                               mask=need)
                            needs.append(need)
                            n += 1
                    out = []
                    n = 0
                    for (msk, idxs, vals) in datas:
                        for c in range(CPT):
                            cur2 = plsc.load_gather(acc_a, [idxs[c]])
                            out.append(needs[n] & (cur2 < vals[c]))
                            n += 1
                    return tuple(out)

                lost = step(None)

                def wcond(st):
                    m = st[0]
                    for s in st[1:]:
                        m = m | s
                    return jnp.any(m)

                lax.while_loop(wcond, step, lost)
                return i + 1

            lax.while_loop(lambda i: jnp.any(i * (16 * ILV) < cnt), pair, 0)

        return max_process

    with jax.named_scope("sc_sums"):
        pltpu.sync_copy(eh_hbm.at[wid], tbl_a)
        pltpu.sync_copy(p_hbm.at[wid], tbl_b)
        zero(acc_a, ACCW)
        zero(acc_b, ACCW)
        stream_edges(g1_hbm, sum_process)
        pltpu.sync_copy(acc_a, den_hbm.at[wid])
        pltpu.sync_copy(acc_b, num_hbm.at[wid])

    with jax.named_scope("sc_max"):
        pltpu.sync_copy(o_hbm.at[wid], tbl_a)
        zero(acc_a, ACCW)
        stream_edges(g1_hbm, make_max_process(NU))
        stream_edges(g2_hbm, make_max_process(0))
        pltpu.sync_copy(acc_a, off_hbm.at[wid])


_sc_call = pl.kernel(
    _sc_body,
    out_type=(
        jax.ShapeDtypeStruct((NTILES, ACCW), jnp.float32),
        jax.ShapeDtypeStruct((NTILES, ACCW), jnp.float32),
        jax.ShapeDtypeStruct((NTILES, ACCW), jnp.float32),
    ),
    mesh=plsc.VectorSubcoreMesh(core_axis_name="c", subcore_axis_name="s"),
    compiler_params=pltpu.CompilerParams(needs_layout_passes=False),
    scratch_types=[
        pltpu.VMEM((TBLW,), jnp.float32),
        pltpu.VMEM((TBLW,), jnp.float32),
        pltpu.VMEM((ACCW,), jnp.float32),
        pltpu.VMEM((ACCW,), jnp.float32),
        pltpu.VMEM((2 * CHUNK,), jnp.int32),
        pltpu.VMEM((2 * CHUNK,), jnp.int32),
        pltpu.SemaphoreType.DMA,
        pltpu.SemaphoreType.DMA,
    ],
)


def _slab(x):
    return x.reshape(NN, NTILES, CPT).transpose(1, 0, 2).reshape(NTILES, TBLW)


def _unslab(x):
    return x.reshape(NTILES, NU, CPT).transpose(1, 0, 2).reshape(NU, D)


def _pack_edges(g):
    return (g.reshape(2, NCH, CHUNK).transpose(1, 0, 2)
             .reshape(NCH, 2 * CHUNK))


def kernel(user_center, user_offset, item_center, item_offset, tag_center,
           tag_offset, graph1, graph2, visit_time, Wc1, bc1, Wc2, bc2,
           Wt1, bt1, Wt2, bt2):
    all_center = jnp.concatenate([user_center, item_center, tag_center], axis=0)
    all_offset = jnp.concatenate([user_offset, item_offset, tag_offset], axis=0)

    eh, p, oo = _tc_pre(all_center, all_offset,
                        Wc1.T, bc1.reshape(1, D),
                        Wc2.T, bc2.reshape(1, D))

    den_s, num_s, off_s = _sc_call(
        _slab(eh), _slab(p), _slab(oo),
        _pack_edges(graph1), _pack_edges(graph2))

    emb, off = _tc_post(_unslab(num_s), _unslab(den_s), _unslab(off_s))
    return emb, off
